# Initial kernel scaffold; baseline (speedup 1.0000x reference)
import jax
import jax.numpy as jnp
from jax.experimental import pallas as pl


def kernel(pcl_noisy, pcl_clean, pnt_idx, fW1, fb1, fW2, fb2, sW1, sb1, sW2, sb2, sW3, sb3):
    raise NotImplementedError("write your pallas kernel here")


# speedup vs baseline: 18.5435x; 18.5435x over previous
import functools

import jax
import jax.numpy as jnp
from jax.experimental import pallas as pl
from jax.experimental.pallas import tpu as pltpu

_FRAME_KNN = 32
_NUM_CLEAN_NBS = 4
_DSM_SIGMA = 0.01
_HIGH = jax.lax.Precision.HIGHEST
_BIG = 3.0e38


def _dn_body(pn_ref, pc_ref, idx_ref, fW1_ref, fb1_ref, fW2_ref, fb2_ref,
             sW1x_ref, sW1z_ref, sb1_ref, sW2_ref, sb2_ref, sW3_ref, sb3_ref,
             out_ref, F_s, T_s, D_s, *, B, N, M, H, RC):
    b = pl.program_id(0)
    K = _FRAME_KNN
    pn = pn_ref[0]
    pc = pc_ref[0]
    idx = idx_ref[...]

    iota = jax.lax.broadcasted_iota(jnp.int32, (M, N), 1)
    oh = (iota == idx).astype(jnp.float32)
    q = jax.lax.dot_general(oh, pn, (((1,), (1,)), ((), ())),
                            precision=_HIGH)

    h = jnp.maximum(jnp.dot(q, fW1_ref[...], precision=_HIGH)
                    + fb1_ref[...], 0.0)
    feat = jnp.dot(h, fW2_ref[...], precision=_HIGH) + fb2_ref[...]
    zpart = jnp.dot(feat, sW1z_ref[...], precision=_HIGH)

    qn = jnp.sum(q * q, axis=1, keepdims=True)
    pnn = jnp.sum(pn * pn, axis=0, keepdims=True)
    cross = jax.lax.dot_general(q, pn, (((1,), (0,)), ((), ())),
                                precision=_HIGH)
    D_s[0:M, :] = qn + pnn - 2.0 * cross

    def frame_step(k, _):
        dd = D_s[0:M, :]
        mn = jnp.min(dd, axis=1, keepdims=True)
        sel = (dd <= mn)
        pt = jax.lax.dot_general(sel.astype(jnp.float32), pn,
                                 (((1,), (1,)), ((), ())),
                                 precision=_HIGH)
        F_s[pl.ds(k * M, M), :] = pt
        D_s[0:M, :] = jnp.where(sel, _BIG, dd)
        return 0

    jax.lax.fori_loop(0, K, frame_step, 0)

    pcn = jnp.sum(pc * pc, axis=0, keepdims=True)

    def clean_step(c, _):
        X = F_s[pl.ds(c * RC, RC), :]
        xn = jnp.sum(X * X, axis=1, keepdims=True)
        cr = jax.lax.dot_general(X, pc, (((1,), (0,)), ((), ())),
                                 precision=_HIGH)
        dd = xn + pcn - 2.0 * cr
        m = jnp.min(dd, axis=1, keepdims=True)
        for _i in range(_NUM_CLEAN_NBS - 1):
            m = jnp.min(jnp.where(dd > m, dd, _BIG), axis=1, keepdims=True)
        mask = (dd <= m).astype(jnp.float32)
        s4 = jax.lax.dot_general(mask, pc, (((1,), (1,)), ((), ())),
                                 precision=_HIGH)
        T_s[pl.ds(c * RC, RC), :] = s4 * (1.0 / _NUM_CLEAN_NBS) - X
        return 0

    jax.lax.fori_loop(0, (M * K) // RC, clean_step, 0)

    Fa = F_s[...]
    Qt = jnp.broadcast_to(q[None], (K, M, 3)).reshape(K * M, 3)
    XC = Fa - Qt
    Zt = jnp.broadcast_to(zpart[None], (K, M, H)).reshape(K * M, H)
    w = sW1x_ref[...]
    h1 = (XC[:, 0:1] * w[0:1, :] + XC[:, 1:2] * w[1:2, :]
          + XC[:, 2:3] * w[2:3, :] + Zt + sb1_ref[...])
    h1 = jnp.maximum(h1, 0.0)
    h2 = jnp.maximum(jnp.dot(h1, sW2_ref[...], precision=_HIGH)
                     + sb2_ref[...], 0.0)
    g = jax.lax.dot_general(h2, sW3_ref[...], (((1,), (0,)), ((), ())),
                            precision=_HIGH) + sb3_ref[...]

    diff = T_s[...] - g
    s = jnp.sum(diff * diff)

    @pl.when(b == 0)
    def _init():
        out_ref[0, 0] = 0.0

    out_ref[0, 0] = out_ref[0, 0] + s

    @pl.when(b == B - 1)
    def _fin():
        scale = 0.5 * (1.0 / _DSM_SIGMA) / (B * M * K)
        out_ref[0, 0] = out_ref[0, 0] * scale


def kernel(pcl_noisy, pcl_clean, pnt_idx, fW1, fb1, fW2, fb2,
           sW1, sb1, sW2, sb2, sW3, sb3):
    B, N, d = pcl_noisy.shape
    M = pnt_idx.shape[0]
    H = fW1.shape[1]
    K = _FRAME_KNN
    RC = 512

    pn_t = jnp.transpose(pcl_noisy, (0, 2, 1))
    pc_t = jnp.transpose(pcl_clean, (0, 2, 1))
    idx2 = pnt_idx.reshape(M, 1).astype(jnp.int32)

    body = functools.partial(_dn_body, B=B, N=N, M=M, H=H, RC=RC)
    bspec = lambda blk, imap: pl.BlockSpec(blk, imap)
    rep = lambda *shape: pl.BlockSpec(shape, lambda b: (0,) * len(shape))

    res = pl.pallas_call(
        body,
        grid=(B,),
        in_specs=[
            bspec((1, 3, N), lambda b: (b, 0, 0)),
            bspec((1, 3, N), lambda b: (b, 0, 0)),
            rep(M, 1),
            rep(3, H), rep(1, H), rep(H, H), rep(1, H),
            rep(3, H), rep(H, H), rep(1, H),
            rep(H, H), rep(1, H), rep(H, 3), rep(1, 3),
        ],
        out_specs=pl.BlockSpec((1, 1), lambda b: (0, 0),
                               memory_space=pltpu.SMEM),
        out_shape=jax.ShapeDtypeStruct((1, 1), jnp.float32),
        scratch_shapes=[
            pltpu.VMEM((M * K, 3), jnp.float32),
            pltpu.VMEM((M * K, 3), jnp.float32),
            pltpu.VMEM((M, N), jnp.float32),
        ],
    )(pn_t, pc_t, idx2,
      fW1, fb1.reshape(1, H), fW2, fb2.reshape(1, H),
      sW1[:d], sW1[d:], sb1.reshape(1, H),
      sW2, sb2.reshape(1, H), sW3, sb3.reshape(1, 3))
    return res[0, 0]


# speedup vs baseline: 19.4863x; 1.0508x over previous
import functools

import jax
import jax.numpy as jnp
from jax.experimental import pallas as pl
from jax.experimental.pallas import tpu as pltpu

_FRAME_KNN = 32
_NUM_CLEAN_NBS = 4
_DSM_SIGMA = 0.01
_HIGH = jax.lax.Precision.HIGHEST
_BIG = 3.0e38


def _dn_body(pn_ref, pc_ref, idx_ref, fW1_ref, fb1_ref, fW2_ref, fb2_ref,
             sW1x_ref, sW1z_ref, sb1_ref, sW2_ref, sb2_ref, sW3_ref, sb3_ref,
             out_ref, F_s, T_s, D_s, *, B, N, M, H, RC):
    b = pl.program_id(0)
    K = _FRAME_KNN
    pn = pn_ref[0]
    pc = pc_ref[0]
    idx = idx_ref[...]

    iota = jax.lax.broadcasted_iota(jnp.int32, (M, N), 1)
    oh = (iota == idx).astype(jnp.float32)
    q = jax.lax.dot_general(oh, pn, (((1,), (1,)), ((), ())),
                            precision=_HIGH)

    h = jnp.maximum(jnp.dot(q, fW1_ref[...]) + fb1_ref[...], 0.0)
    feat = jnp.dot(h, fW2_ref[...]) + fb2_ref[...]
    zpart = jnp.dot(feat, sW1z_ref[...])

    qn = jnp.sum(q * q, axis=1, keepdims=True)
    pnn = jnp.sum(pn * pn, axis=0, keepdims=True)
    cross = jax.lax.dot_general(q, pn, (((1,), (0,)), ((), ())),
                                precision=_HIGH)
    D_s[0:M, :] = qn + pnn - 2.0 * cross

    def frame_step(k, _):
        dd = D_s[0:M, :]
        mn = jnp.min(dd, axis=1, keepdims=True)
        sel = (dd <= mn)
        pt = jax.lax.dot_general(sel.astype(jnp.float32), pn,
                                 (((1,), (1,)), ((), ())),
                                 precision=_HIGH)
        F_s[pl.ds(k * M, M), :] = pt
        D_s[0:M, :] = jnp.where(sel, _BIG, dd)
        return 0

    jax.lax.fori_loop(0, K, frame_step, 0)

    pcn = jnp.sum(pc * pc, axis=0, keepdims=True)

    def clean_step(c, _):
        X = F_s[pl.ds(c * RC, RC), :]
        xn = jnp.sum(X * X, axis=1, keepdims=True)
        cr = jax.lax.dot_general(X, pc, (((1,), (0,)), ((), ())),
                                 precision=_HIGH)
        dd = xn + pcn - 2.0 * cr
        m = jnp.min(dd, axis=1, keepdims=True)
        for _i in range(_NUM_CLEAN_NBS - 1):
            m = jnp.min(jnp.where(dd > m, dd, _BIG), axis=1, keepdims=True)
        mask = (dd <= m).astype(jnp.float32)
        s4 = jax.lax.dot_general(mask, pc, (((1,), (1,)), ((), ())),
                                 precision=_HIGH)
        T_s[pl.ds(c * RC, RC), :] = s4 * (1.0 / _NUM_CLEAN_NBS) - X
        return 0

    jax.lax.fori_loop(0, (M * K) // RC, clean_step, 0)

    Fa = F_s[...]
    Qt = jnp.broadcast_to(q[None], (K, M, 3)).reshape(K * M, 3)
    XC = Fa - Qt
    Zt = jnp.broadcast_to(zpart[None], (K, M, H)).reshape(K * M, H)
    w = sW1x_ref[...]
    h1 = (XC[:, 0:1] * w[0:1, :] + XC[:, 1:2] * w[1:2, :]
          + XC[:, 2:3] * w[2:3, :] + Zt + sb1_ref[...])
    h1 = jnp.maximum(h1, 0.0)
    h2 = jnp.maximum(jnp.dot(h1, sW2_ref[...]) + sb2_ref[...], 0.0)
    g = jax.lax.dot_general(h2, sW3_ref[...],
                            (((1,), (0,)), ((), ()))) + sb3_ref[...]

    diff = T_s[...] - g
    s = jnp.sum(diff * diff)

    @pl.when(b == 0)
    def _init():
        out_ref[0, 0] = 0.0

    out_ref[0, 0] = out_ref[0, 0] + s

    @pl.when(b == B - 1)
    def _fin():
        scale = 0.5 * (1.0 / _DSM_SIGMA) / (B * M * K)
        out_ref[0, 0] = out_ref[0, 0] * scale


def kernel(pcl_noisy, pcl_clean, pnt_idx, fW1, fb1, fW2, fb2,
           sW1, sb1, sW2, sb2, sW3, sb3):
    B, N, d = pcl_noisy.shape
    M = pnt_idx.shape[0]
    H = fW1.shape[1]
    K = _FRAME_KNN
    RC = 1024

    pn_t = jnp.transpose(pcl_noisy, (0, 2, 1))
    pc_t = jnp.transpose(pcl_clean, (0, 2, 1))
    idx2 = pnt_idx.reshape(M, 1).astype(jnp.int32)

    body = functools.partial(_dn_body, B=B, N=N, M=M, H=H, RC=RC)
    bspec = lambda blk, imap: pl.BlockSpec(blk, imap)
    rep = lambda *shape: pl.BlockSpec(shape, lambda b: (0,) * len(shape))

    res = pl.pallas_call(
        body,
        grid=(B,),
        in_specs=[
            bspec((1, 3, N), lambda b: (b, 0, 0)),
            bspec((1, 3, N), lambda b: (b, 0, 0)),
            rep(M, 1),
            rep(3, H), rep(1, H), rep(H, H), rep(1, H),
            rep(3, H), rep(H, H), rep(1, H),
            rep(H, H), rep(1, H), rep(H, 3), rep(1, 3),
        ],
        out_specs=pl.BlockSpec((1, 1), lambda b: (0, 0),
                               memory_space=pltpu.SMEM),
        out_shape=jax.ShapeDtypeStruct((1, 1), jnp.float32),
        scratch_shapes=[
            pltpu.VMEM((M * K, 3), jnp.float32),
            pltpu.VMEM((M * K, 3), jnp.float32),
            pltpu.VMEM((M, N), jnp.float32),
        ],
    )(pn_t, pc_t, idx2,
      fW1, fb1.reshape(1, H), fW2, fb2.reshape(1, H),
      sW1[:d], sW1[d:], sb1.reshape(1, H),
      sW2, sb2.reshape(1, H), sW3, sb3.reshape(1, 3))
    return res[0, 0]


# speedup vs baseline: 46.4759x; 2.3851x over previous
import functools

import jax
import jax.numpy as jnp
from jax.experimental import pallas as pl
from jax.experimental.pallas import tpu as pltpu

_FRAME_KNN = 32
_NUM_CLEAN_NBS = 4
_DSM_SIGMA = 0.01
_HIGH = jax.lax.Precision.HIGHEST
_BIG = 3.0e38


def _dn_body(pn_ref, pc_ref, pnr_ref, idx_ref, fW1_ref, fb1_ref, fW2_ref,
             fb2_ref, sW1x_ref, sW1z_ref, sb1_ref, sW2_ref, sb2_ref, sW3_ref,
             sb3_ref, out_ref, F_s, T_s, D_s, THi_s, TLo_s, *, B, N, M, H, RC):
    b = pl.program_id(0)
    K = _FRAME_KNN
    pn = pn_ref[0]
    pc = pc_ref[0]
    idx = idx_ref[...]

    pn_hi = pn.astype(jnp.bfloat16)
    pn_lo = (pn - pn_hi.astype(jnp.float32)).astype(jnp.bfloat16)
    pc_hi = pc.astype(jnp.bfloat16)
    pc_lo = (pc - pc_hi.astype(jnp.float32)).astype(jnp.bfloat16)

    def extract2(selb, rhs_hi, rhs_lo, dims):
        acc = jax.lax.dot_general(selb, rhs_hi, (dims, ((), ())),
                                  preferred_element_type=jnp.float32)
        return acc + jax.lax.dot_general(selb, rhs_lo, (dims, ((), ())),
                                         preferred_element_type=jnp.float32)

    iota = jax.lax.broadcasted_iota(jnp.int32, (M, N), 1)
    ohb = (iota == idx).astype(jnp.bfloat16)
    q = extract2(ohb, pn_hi, pn_lo, ((1,), (1,)))

    h = jnp.maximum(jnp.dot(q, fW1_ref[...]) + fb1_ref[...], 0.0)
    feat = jnp.dot(h, fW2_ref[...]) + fb2_ref[...]
    zpart = jnp.dot(feat, sW1z_ref[...])

    pcn = jnp.sum(pc * pc, axis=0, keepdims=True)

    def clean_step(c, _):
        X = pnr_ref[0, pl.ds(c * RC, RC), :]
        xn = jnp.sum(X * X, axis=1, keepdims=True)
        cr = jax.lax.dot_general(X, pc, (((1,), (0,)), ((), ())),
                                 precision=_HIGH)
        dd = xn + pcn - 2.0 * cr
        m = jnp.min(dd, axis=1, keepdims=True)
        for _i in range(_NUM_CLEAN_NBS - 1):
            m = jnp.min(jnp.where(dd > m, dd, _BIG), axis=1, keepdims=True)
        mask = (dd <= m).astype(jnp.bfloat16)
        s4 = extract2(mask, pc_hi, pc_lo, ((1,), (1,)))
        tgt = s4 * (1.0 / _NUM_CLEAN_NBS) - X
        thi = tgt.astype(jnp.bfloat16)
        THi_s[pl.ds(c * RC, RC), :] = thi
        TLo_s[pl.ds(c * RC, RC), :] = (tgt - thi.astype(jnp.float32)
                                       ).astype(jnp.bfloat16)
        return 0

    jax.lax.fori_loop(0, N // RC, clean_step, 0)

    qn = jnp.sum(q * q, axis=1, keepdims=True)
    pnn = jnp.sum(pn * pn, axis=0, keepdims=True)
    cross = jax.lax.dot_general(q, pn, (((1,), (0,)), ((), ())),
                                precision=_HIGH)
    D_s[0:M, :] = qn + pnn - 2.0 * cross

    def frame_step(k, _):
        dd = D_s[0:M, :]
        mn = jnp.min(dd, axis=1, keepdims=True)
        sel = (dd <= mn)
        selb = sel.astype(jnp.bfloat16)
        F_s[pl.ds(k * M, M), :] = extract2(selb, pn_hi, pn_lo, ((1,), (1,)))
        T_s[pl.ds(k * M, M), :] = extract2(selb, THi_s[...], TLo_s[...],
                                           ((1,), (0,)))
        D_s[0:M, :] = jnp.where(sel, _BIG, dd)
        return 0

    jax.lax.fori_loop(0, K, frame_step, 0)

    Fa = F_s[...]
    Qt = jnp.broadcast_to(q[None], (K, M, 3)).reshape(K * M, 3)
    XC = Fa - Qt
    Zt = jnp.broadcast_to(zpart[None], (K, M, H)).reshape(K * M, H)
    w = sW1x_ref[...]
    h1 = (XC[:, 0:1] * w[0:1, :] + XC[:, 1:2] * w[1:2, :]
          + XC[:, 2:3] * w[2:3, :] + Zt + sb1_ref[...])
    h1 = jnp.maximum(h1, 0.0)
    h2 = jnp.maximum(jnp.dot(h1, sW2_ref[...]) + sb2_ref[...], 0.0)
    g = jax.lax.dot_general(h2, sW3_ref[...],
                            (((1,), (0,)), ((), ()))) + sb3_ref[...]

    diff = T_s[...] - g
    s = jnp.sum(diff * diff)

    @pl.when(b == 0)
    def _init():
        out_ref[0, 0] = 0.0

    out_ref[0, 0] = out_ref[0, 0] + s

    @pl.when(b == B - 1)
    def _fin():
        scale = 0.5 * (1.0 / _DSM_SIGMA) / (B * M * K)
        out_ref[0, 0] = out_ref[0, 0] * scale


def kernel(pcl_noisy, pcl_clean, pnt_idx, fW1, fb1, fW2, fb2,
           sW1, sb1, sW2, sb2, sW3, sb3):
    B, N, d = pcl_noisy.shape
    M = pnt_idx.shape[0]
    H = fW1.shape[1]
    K = _FRAME_KNN
    RC = 1024

    pn_t = jnp.transpose(pcl_noisy, (0, 2, 1))
    pc_t = jnp.transpose(pcl_clean, (0, 2, 1))
    idx2 = pnt_idx.reshape(M, 1).astype(jnp.int32)

    body = functools.partial(_dn_body, B=B, N=N, M=M, H=H, RC=RC)
    bspec = lambda blk, imap: pl.BlockSpec(blk, imap)
    rep = lambda *shape: pl.BlockSpec(shape, lambda b: (0,) * len(shape))

    res = pl.pallas_call(
        body,
        grid=(B,),
        in_specs=[
            bspec((1, 3, N), lambda b: (b, 0, 0)),
            bspec((1, 3, N), lambda b: (b, 0, 0)),
            bspec((1, N, 3), lambda b: (b, 0, 0)),
            rep(M, 1),
            rep(3, H), rep(1, H), rep(H, H), rep(1, H),
            rep(3, H), rep(H, H), rep(1, H),
            rep(H, H), rep(1, H), rep(H, 3), rep(1, 3),
        ],
        out_specs=pl.BlockSpec((1, 1), lambda b: (0, 0),
                               memory_space=pltpu.SMEM),
        out_shape=jax.ShapeDtypeStruct((1, 1), jnp.float32),
        scratch_shapes=[
            pltpu.VMEM((M * K, 3), jnp.float32),
            pltpu.VMEM((M * K, 3), jnp.float32),
            pltpu.VMEM((M, N), jnp.float32),
            pltpu.VMEM((N, 3), jnp.bfloat16),
            pltpu.VMEM((N, 3), jnp.bfloat16),
        ],
    )(pn_t, pc_t, pcl_noisy, idx2,
      fW1, fb1.reshape(1, H), fW2, fb2.reshape(1, H),
      sW1[:d], sW1[d:], sb1.reshape(1, H),
      sW2, sb2.reshape(1, H), sW3, sb3.reshape(1, 3))
    return res[0, 0]


# speedup vs baseline: 58.0554x; 1.2492x over previous
import functools

import jax
import jax.numpy as jnp
from jax.experimental import pallas as pl
from jax.experimental.pallas import tpu as pltpu

_FRAME_KNN = 32
_NUM_CLEAN_NBS = 4
_DSM_SIGMA = 0.01
_HIGH = jax.lax.Precision.HIGHEST
_BIG = 3.0e38


def _dn_body(pn_ref, pc_ref, pnr_ref, idx_ref, fW1_ref, fb1_ref, fW2_ref,
             fb2_ref, sW1x_ref, sW1z_ref, sb1_ref, sW2_ref, sb2_ref, sW3_ref,
             sb3_ref, out_ref, F_s, T_s, D_s, THi_s, TLo_s, *, B, N, M, H, RC):
    b = pl.program_id(0)
    K = _FRAME_KNN
    pn = pn_ref[0]
    pc = pc_ref[0]
    idx = idx_ref[...]

    pn_hi = pn.astype(jnp.bfloat16)
    pn_lo = (pn - pn_hi.astype(jnp.float32)).astype(jnp.bfloat16)
    pc_hi = pc.astype(jnp.bfloat16)
    pc_lo = (pc - pc_hi.astype(jnp.float32)).astype(jnp.bfloat16)

    def extract2(selb, rhs_hi, rhs_lo, dims):
        acc = jax.lax.dot_general(selb, rhs_hi, (dims, ((), ())),
                                  preferred_element_type=jnp.float32)
        return acc + jax.lax.dot_general(selb, rhs_lo, (dims, ((), ())),
                                         preferred_element_type=jnp.float32)

    def split(a):
        hi = a.astype(jnp.bfloat16)
        return hi, (a - hi.astype(jnp.float32)).astype(jnp.bfloat16)

    def cross3(ahi, alo, bhi, blo, dims):
        dn = (dims, ((), ()))
        return (jax.lax.dot_general(ahi, bhi, dn,
                                    preferred_element_type=jnp.float32)
                + jax.lax.dot_general(ahi, blo, dn,
                                      preferred_element_type=jnp.float32)
                + jax.lax.dot_general(alo, bhi, dn,
                                      preferred_element_type=jnp.float32))

    iota = jax.lax.broadcasted_iota(jnp.int32, (M, N), 1)
    ohb = (iota == idx).astype(jnp.bfloat16)
    q = extract2(ohb, pn_hi, pn_lo, ((1,), (1,)))

    h = jnp.maximum(jnp.dot(q, fW1_ref[...]) + fb1_ref[...], 0.0)
    feat = jnp.dot(h, fW2_ref[...]) + fb2_ref[...]
    zpart = jnp.dot(feat, sW1z_ref[...])

    pcn = jnp.sum(pc * pc, axis=0, keepdims=True)

    def clean_step(c, _):
        X = pnr_ref[0, pl.ds(c * RC, RC), :]
        xn = jnp.sum(X * X, axis=1, keepdims=True)
        xhi, xlo = split(X)
        cr = cross3(xhi, xlo, pc_hi, pc_lo, ((1,), (0,)))
        dd = xn + pcn - 2.0 * cr
        m = jnp.min(dd, axis=1, keepdims=True)
        for _i in range(_NUM_CLEAN_NBS - 1):
            m = jnp.min(jnp.where(dd > m, dd, _BIG), axis=1, keepdims=True)
        mask = (dd <= m).astype(jnp.bfloat16)
        s4 = extract2(mask, pc_hi, pc_lo, ((1,), (1,)))
        tgt = s4 * (1.0 / _NUM_CLEAN_NBS) - X
        val = jnp.concatenate([X, tgt], axis=1)
        vhi, vlo = split(val)
        THi_s[pl.ds(c * RC, RC), :] = vhi
        TLo_s[pl.ds(c * RC, RC), :] = vlo
        return 0

    jax.lax.fori_loop(0, N // RC, clean_step, 0)

    qn = jnp.sum(q * q, axis=1, keepdims=True)
    pnn = jnp.sum(pn * pn, axis=0, keepdims=True)
    qhi, qlo = split(q)
    cross = cross3(qhi, qlo, pn_hi, pn_lo, ((1,), (0,)))
    D_s[0:M, :] = qn + pnn - 2.0 * cross

    def frame_step(k, _):
        dd = D_s[0:M, :]
        mn = jnp.min(dd, axis=1, keepdims=True)
        sel = (dd <= mn)
        selb = sel.astype(jnp.bfloat16)
        ext = extract2(selb, THi_s[...], TLo_s[...], ((1,), (0,)))
        F_s[pl.ds(k * M, M), :] = ext[:, 0:3]
        T_s[pl.ds(k * M, M), :] = ext[:, 3:6]
        D_s[0:M, :] = jnp.where(sel, _BIG, dd)
        return 0

    jax.lax.fori_loop(0, K, frame_step, 0)

    Fa = F_s[...]
    Qt = jnp.broadcast_to(q[None], (K, M, 3)).reshape(K * M, 3)
    XC = Fa - Qt
    Zt = jnp.broadcast_to(zpart[None], (K, M, H)).reshape(K * M, H)
    w = sW1x_ref[...]
    h1 = (XC[:, 0:1] * w[0:1, :] + XC[:, 1:2] * w[1:2, :]
          + XC[:, 2:3] * w[2:3, :] + Zt + sb1_ref[...])
    h1 = jnp.maximum(h1, 0.0)
    h2 = jnp.maximum(jnp.dot(h1, sW2_ref[...]) + sb2_ref[...], 0.0)
    g = jax.lax.dot_general(h2, sW3_ref[...],
                            (((1,), (0,)), ((), ()))) + sb3_ref[...]

    diff = T_s[...] - g
    s = jnp.sum(diff * diff)

    @pl.when(b == 0)
    def _init():
        out_ref[0, 0] = 0.0

    out_ref[0, 0] = out_ref[0, 0] + s

    @pl.when(b == B - 1)
    def _fin():
        scale = 0.5 * (1.0 / _DSM_SIGMA) / (B * M * K)
        out_ref[0, 0] = out_ref[0, 0] * scale


def kernel(pcl_noisy, pcl_clean, pnt_idx, fW1, fb1, fW2, fb2,
           sW1, sb1, sW2, sb2, sW3, sb3):
    B, N, d = pcl_noisy.shape
    M = pnt_idx.shape[0]
    H = fW1.shape[1]
    K = _FRAME_KNN
    RC = 1024

    pn_t = jnp.transpose(pcl_noisy, (0, 2, 1))
    pc_t = jnp.transpose(pcl_clean, (0, 2, 1))
    idx2 = pnt_idx.reshape(M, 1).astype(jnp.int32)

    body = functools.partial(_dn_body, B=B, N=N, M=M, H=H, RC=RC)
    bspec = lambda blk, imap: pl.BlockSpec(blk, imap)
    rep = lambda *shape: pl.BlockSpec(shape, lambda b: (0,) * len(shape))

    res = pl.pallas_call(
        body,
        grid=(B,),
        in_specs=[
            bspec((1, 3, N), lambda b: (b, 0, 0)),
            bspec((1, 3, N), lambda b: (b, 0, 0)),
            bspec((1, N, 3), lambda b: (b, 0, 0)),
            rep(M, 1),
            rep(3, H), rep(1, H), rep(H, H), rep(1, H),
            rep(3, H), rep(H, H), rep(1, H),
            rep(H, H), rep(1, H), rep(H, 3), rep(1, 3),
        ],
        out_specs=pl.BlockSpec((1, 1), lambda b: (0, 0),
                               memory_space=pltpu.SMEM),
        out_shape=jax.ShapeDtypeStruct((1, 1), jnp.float32),
        scratch_shapes=[
            pltpu.VMEM((M * K, 3), jnp.float32),
            pltpu.VMEM((M * K, 3), jnp.float32),
            pltpu.VMEM((M, N), jnp.float32),
            pltpu.VMEM((N, 6), jnp.bfloat16),
            pltpu.VMEM((N, 6), jnp.bfloat16),
        ],
    )(pn_t, pc_t, pcl_noisy, idx2,
      fW1, fb1.reshape(1, H), fW2, fb2.reshape(1, H),
      sW1[:d], sW1[d:], sb1.reshape(1, H),
      sW2, sb2.reshape(1, H), sW3, sb3.reshape(1, 3))
    return res[0, 0]


# speedup vs baseline: 63.2309x; 1.0891x over previous
import functools

import jax
import jax.numpy as jnp
from jax.experimental import pallas as pl
from jax.experimental.pallas import tpu as pltpu

_FRAME_KNN = 32
_NUM_CLEAN_NBS = 4
_DSM_SIGMA = 0.01
_HIGH = jax.lax.Precision.HIGHEST
_BIG = 3.0e38


def _dn_body(pn_ref, pc_ref, pnr_ref, idx_ref, fW1_ref, fb1_ref, fW2_ref,
             fb2_ref, sW1x_ref, sW1z_ref, sb1_ref, sW2_ref, sb2_ref, sW3_ref,
             sb3_ref, out_ref, Sel_s, D_s, THi_s, TLo_s, *, B, N, M, H, RC):
    b = pl.program_id(0)
    K = _FRAME_KNN
    pn = pn_ref[0]
    pc = pc_ref[0]
    idx = idx_ref[...]

    pn_hi = pn.astype(jnp.bfloat16)
    pn_lo = (pn - pn_hi.astype(jnp.float32)).astype(jnp.bfloat16)
    pc_hi = pc.astype(jnp.bfloat16)
    pc_lo = (pc - pc_hi.astype(jnp.float32)).astype(jnp.bfloat16)

    def extract2(selb, rhs_hi, rhs_lo, dims):
        acc = jax.lax.dot_general(selb, rhs_hi, (dims, ((), ())),
                                  preferred_element_type=jnp.float32)
        return acc + jax.lax.dot_general(selb, rhs_lo, (dims, ((), ())),
                                         preferred_element_type=jnp.float32)

    def split(a):
        hi = a.astype(jnp.bfloat16)
        return hi, (a - hi.astype(jnp.float32)).astype(jnp.bfloat16)

    def cross3(ahi, alo, bhi, blo, dims):
        dn = (dims, ((), ()))
        return (jax.lax.dot_general(ahi, bhi, dn,
                                    preferred_element_type=jnp.float32)
                + jax.lax.dot_general(ahi, blo, dn,
                                      preferred_element_type=jnp.float32)
                + jax.lax.dot_general(alo, bhi, dn,
                                      preferred_element_type=jnp.float32))

    iota = jax.lax.broadcasted_iota(jnp.int32, (M, N), 1)
    ohb = (iota == idx).astype(jnp.bfloat16)
    q = extract2(ohb, pn_hi, pn_lo, ((1,), (1,)))

    h = jnp.maximum(jnp.dot(q, fW1_ref[...]) + fb1_ref[...], 0.0)
    feat = jnp.dot(h, fW2_ref[...]) + fb2_ref[...]
    zpart = jnp.dot(feat, sW1z_ref[...])

    pcn = jnp.sum(pc * pc, axis=0, keepdims=True)

    def clean_step(c, _):
        X = pnr_ref[0, pl.ds(c * RC, RC), :]
        xn = jnp.sum(X * X, axis=1, keepdims=True)
        xhi, xlo = split(X)
        cr = cross3(xhi, xlo, pc_hi, pc_lo, ((1,), (0,)))
        dd = xn + pcn - 2.0 * cr
        m = jnp.min(dd, axis=1, keepdims=True)
        for _i in range(_NUM_CLEAN_NBS - 1):
            m = jnp.min(jnp.where(dd > m, dd, _BIG), axis=1, keepdims=True)
        mask = (dd <= m).astype(jnp.bfloat16)
        s4 = extract2(mask, pc_hi, pc_lo, ((1,), (1,)))
        tgt = s4 * (1.0 / _NUM_CLEAN_NBS) - X
        val = jnp.concatenate([X, tgt], axis=1)
        vhi, vlo = split(val)
        THi_s[pl.ds(c * RC, RC), :] = vhi
        TLo_s[pl.ds(c * RC, RC), :] = vlo
        return 0

    jax.lax.fori_loop(0, N // RC, clean_step, 0)

    qn = jnp.sum(q * q, axis=1, keepdims=True)
    pnn = jnp.sum(pn * pn, axis=0, keepdims=True)
    qhi, qlo = split(q)
    cross = cross3(qhi, qlo, pn_hi, pn_lo, ((1,), (0,)))
    D_s[0:M, :] = qn + pnn - 2.0 * cross

    def frame_step(k, _):
        dd = D_s[0:M, :]
        mn = jnp.min(dd, axis=1, keepdims=True)
        sel = (dd <= mn)
        Sel_s[pl.ds(k * M, M), :] = sel.astype(jnp.bfloat16)
        D_s[0:M, :] = jnp.where(sel, _BIG, dd)
        return 0

    jax.lax.fori_loop(0, K, frame_step, 0)

    ext = extract2(Sel_s[...], THi_s[...], TLo_s[...], ((1,), (0,)))

    Fa = ext[:, 0:3]
    Qt = jnp.broadcast_to(q[None], (K, M, 3)).reshape(K * M, 3)
    XC = Fa - Qt
    Zt = jnp.broadcast_to(zpart[None], (K, M, H)).reshape(K * M, H)
    w = sW1x_ref[...]
    h1 = (XC[:, 0:1] * w[0:1, :] + XC[:, 1:2] * w[1:2, :]
          + XC[:, 2:3] * w[2:3, :] + Zt + sb1_ref[...])
    h1 = jnp.maximum(h1, 0.0)
    h2 = jnp.maximum(jnp.dot(h1, sW2_ref[...]) + sb2_ref[...], 0.0)
    g = jax.lax.dot_general(h2, sW3_ref[...],
                            (((1,), (0,)), ((), ()))) + sb3_ref[...]

    diff = ext[:, 3:6] - g
    s = jnp.sum(diff * diff)

    @pl.when(b == 0)
    def _init():
        out_ref[0, 0] = 0.0

    out_ref[0, 0] = out_ref[0, 0] + s

    @pl.when(b == B - 1)
    def _fin():
        scale = 0.5 * (1.0 / _DSM_SIGMA) / (B * M * K)
        out_ref[0, 0] = out_ref[0, 0] * scale


def kernel(pcl_noisy, pcl_clean, pnt_idx, fW1, fb1, fW2, fb2,
           sW1, sb1, sW2, sb2, sW3, sb3):
    B, N, d = pcl_noisy.shape
    M = pnt_idx.shape[0]
    H = fW1.shape[1]
    K = _FRAME_KNN
    RC = 1024

    pn_t = jnp.transpose(pcl_noisy, (0, 2, 1))
    pc_t = jnp.transpose(pcl_clean, (0, 2, 1))
    idx2 = pnt_idx.reshape(M, 1).astype(jnp.int32)

    body = functools.partial(_dn_body, B=B, N=N, M=M, H=H, RC=RC)
    bspec = lambda blk, imap: pl.BlockSpec(blk, imap)
    rep = lambda *shape: pl.BlockSpec(shape, lambda b: (0,) * len(shape))

    res = pl.pallas_call(
        body,
        grid=(B,),
        in_specs=[
            bspec((1, 3, N), lambda b: (b, 0, 0)),
            bspec((1, 3, N), lambda b: (b, 0, 0)),
            bspec((1, N, 3), lambda b: (b, 0, 0)),
            rep(M, 1),
            rep(3, H), rep(1, H), rep(H, H), rep(1, H),
            rep(3, H), rep(H, H), rep(1, H),
            rep(H, H), rep(1, H), rep(H, 3), rep(1, 3),
        ],
        out_specs=pl.BlockSpec((1, 1), lambda b: (0, 0),
                               memory_space=pltpu.SMEM),
        out_shape=jax.ShapeDtypeStruct((1, 1), jnp.float32),
        scratch_shapes=[
            pltpu.VMEM((M * K, N), jnp.bfloat16),
            pltpu.VMEM((M, N), jnp.float32),
            pltpu.VMEM((N, 6), jnp.bfloat16),
            pltpu.VMEM((N, 6), jnp.bfloat16),
        ],
    )(pn_t, pc_t, pcl_noisy, idx2,
      fW1, fb1.reshape(1, H), fW2, fb2.reshape(1, H),
      sW1[:d], sW1[d:], sb1.reshape(1, H),
      sW2, sb2.reshape(1, H), sW3, sb3.reshape(1, 3))
    return res[0, 0]


# speedup vs baseline: 76.0834x; 1.2033x over previous
import functools

import jax
import jax.numpy as jnp
from jax.experimental import pallas as pl
from jax.experimental.pallas import tpu as pltpu

_FRAME_KNN = 32
_NUM_CLEAN_NBS = 4
_DSM_SIGMA = 0.01
_HIGH = jax.lax.Precision.HIGHEST
_BIG = 3.0e38


def _dn_body(pn_ref, pc_ref, pnr_ref, idx_ref, fW1_ref, fb1_ref, fW2_ref,
             fb2_ref, sW1x_ref, sW1z_ref, sb1_ref, sW2_ref, sb2_ref, sW3_ref,
             sb3_ref, out_ref, Sel_s, D_s, THi_s, TLo_s, *, B, N, M, H, RC):
    b = pl.program_id(0)
    K = _FRAME_KNN
    pn = pn_ref[0]
    pc = pc_ref[0]
    idx = idx_ref[...]

    pn_hi = pn.astype(jnp.bfloat16)
    pn_lo = (pn - pn_hi.astype(jnp.float32)).astype(jnp.bfloat16)
    pc_hi = pc.astype(jnp.bfloat16)
    pc_lo = (pc - pc_hi.astype(jnp.float32)).astype(jnp.bfloat16)
    pn9 = jnp.concatenate([pn_hi, pn_lo, pn_hi], axis=0)
    pc9 = jnp.concatenate([pc_hi, pc_lo, pc_hi], axis=0)

    def extract2(selb, rhs_hi, rhs_lo, dims):
        acc = jax.lax.dot_general(selb, rhs_hi, (dims, ((), ())),
                                  preferred_element_type=jnp.float32)
        return acc + jax.lax.dot_general(selb, rhs_lo, (dims, ((), ())),
                                         preferred_element_type=jnp.float32)

    def split(a):
        hi = a.astype(jnp.bfloat16)
        return hi, (a - hi.astype(jnp.float32)).astype(jnp.bfloat16)

    def cross3(ahi, alo, bstack):
        a9 = jnp.concatenate([ahi, ahi, alo], axis=1)
        return jax.lax.dot_general(a9, bstack, (((1,), (0,)), ((), ())),
                                   preferred_element_type=jnp.float32)

    iota = jax.lax.broadcasted_iota(jnp.int32, (M, N), 1)
    ohb = (iota == idx).astype(jnp.bfloat16)
    q = extract2(ohb, pn_hi, pn_lo, ((1,), (1,)))

    h = jnp.maximum(jnp.dot(q, fW1_ref[...]) + fb1_ref[...], 0.0)
    feat = jnp.dot(h, fW2_ref[...]) + fb2_ref[...]
    zpart = jnp.dot(feat, sW1z_ref[...])

    pcn = jnp.sum(pc * pc, axis=0, keepdims=True)

    def clean_step(c, _):
        X = pnr_ref[0, pl.ds(c * RC, RC), :]
        xn = jnp.sum(X * X, axis=1, keepdims=True)
        xhi, xlo = split(X)
        cr = cross3(xhi, xlo, pc9)
        dd = xn + pcn - 2.0 * cr
        m = jnp.min(dd, axis=1, keepdims=True)
        for _i in range(_NUM_CLEAN_NBS - 1):
            m = jnp.min(jnp.where(dd > m, dd, _BIG), axis=1, keepdims=True)
        mask = (dd <= m).astype(jnp.bfloat16)
        s4 = extract2(mask, pc_hi, pc_lo, ((1,), (1,)))
        tgt = s4 * (1.0 / _NUM_CLEAN_NBS) - X
        val = jnp.concatenate([X, tgt], axis=1)
        vhi, vlo = split(val)
        THi_s[pl.ds(c * RC, RC), :] = vhi
        TLo_s[pl.ds(c * RC, RC), :] = vlo
        return 0

    qn = jnp.sum(q * q, axis=1, keepdims=True)
    pnn = jnp.sum(pn * pn, axis=0, keepdims=True)
    qhi, qlo = split(q)
    cross = cross3(qhi, qlo, pn9)
    D_s[0:M, :] = qn + pnn - 2.0 * cross

    def frame_step(k, _):
        dd = D_s[0:M, :]
        mn = jnp.min(dd, axis=1, keepdims=True)
        sel = (dd <= mn)
        Sel_s[pl.ds(k * M, M), :] = sel.astype(jnp.bfloat16)
        D_s[0:M, :] = jnp.where(sel, _BIG, dd)
        return 0

    NCH = N // RC

    def fused_step(k, _):
        frame_step(k, _)
        clean_step(k, _)
        return 0

    jax.lax.fori_loop(0, NCH, fused_step, 0)
    jax.lax.fori_loop(NCH, K, frame_step, 0)

    ext = extract2(Sel_s[...], THi_s[...], TLo_s[...], ((1,), (0,)))

    Fa = ext[:, 0:3]
    Qt = jnp.broadcast_to(q[None], (K, M, 3)).reshape(K * M, 3)
    XC = Fa - Qt
    Zt = jnp.broadcast_to(zpart[None], (K, M, H)).reshape(K * M, H)
    h1 = jnp.dot(XC, sW1x_ref[...]) + Zt + sb1_ref[...]
    h1 = jnp.maximum(h1, 0.0)
    h2 = jnp.maximum(jnp.dot(h1, sW2_ref[...]) + sb2_ref[...], 0.0)
    g = jax.lax.dot_general(h2, sW3_ref[...],
                            (((1,), (0,)), ((), ()))) + sb3_ref[...]

    diff = ext[:, 3:6] - g
    s = jnp.sum(diff * diff)

    @pl.when(b == 0)
    def _init():
        out_ref[0, 0] = 0.0

    out_ref[0, 0] = out_ref[0, 0] + s

    @pl.when(b == B - 1)
    def _fin():
        scale = 0.5 * (1.0 / _DSM_SIGMA) / (B * M * K)
        out_ref[0, 0] = out_ref[0, 0] * scale


def kernel(pcl_noisy, pcl_clean, pnt_idx, fW1, fb1, fW2, fb2,
           sW1, sb1, sW2, sb2, sW3, sb3):
    B, N, d = pcl_noisy.shape
    M = pnt_idx.shape[0]
    H = fW1.shape[1]
    K = _FRAME_KNN
    RC = 256

    pn_t = jnp.transpose(pcl_noisy, (0, 2, 1))
    pc_t = jnp.transpose(pcl_clean, (0, 2, 1))
    idx2 = pnt_idx.reshape(M, 1).astype(jnp.int32)

    body = functools.partial(_dn_body, B=B, N=N, M=M, H=H, RC=RC)
    bspec = lambda blk, imap: pl.BlockSpec(blk, imap)
    rep = lambda *shape: pl.BlockSpec(shape, lambda b: (0,) * len(shape))

    res = pl.pallas_call(
        body,
        grid=(B,),
        in_specs=[
            bspec((1, 3, N), lambda b: (b, 0, 0)),
            bspec((1, 3, N), lambda b: (b, 0, 0)),
            bspec((1, N, 3), lambda b: (b, 0, 0)),
            rep(M, 1),
            rep(3, H), rep(1, H), rep(H, H), rep(1, H),
            rep(3, H), rep(H, H), rep(1, H),
            rep(H, H), rep(1, H), rep(H, 3), rep(1, 3),
        ],
        out_specs=pl.BlockSpec((1, 1), lambda b: (0, 0),
                               memory_space=pltpu.SMEM),
        out_shape=jax.ShapeDtypeStruct((1, 1), jnp.float32),
        scratch_shapes=[
            pltpu.VMEM((M * K, N), jnp.bfloat16),
            pltpu.VMEM((M, N), jnp.float32),
            pltpu.VMEM((N, 6), jnp.bfloat16),
            pltpu.VMEM((N, 6), jnp.bfloat16),
        ],
    )(pn_t, pc_t, pcl_noisy, idx2,
      fW1, fb1.reshape(1, H), fW2, fb2.reshape(1, H),
      sW1[:d], sW1[d:], sb1.reshape(1, H),
      sW2, sb2.reshape(1, H), sW3, sb3.reshape(1, 3))
    return res[0, 0]


# speedup vs baseline: 86.2978x; 1.1343x over previous
import functools

import jax
import jax.numpy as jnp
from jax.experimental import pallas as pl
from jax.experimental.pallas import tpu as pltpu

_FRAME_KNN = 32
_NUM_CLEAN_NBS = 4
_DSM_SIGMA = 0.01
_HIGH = jax.lax.Precision.HIGHEST
_BIG = 3.0e38


def _dn_body(pn_ref, pc_ref, pnr_ref, idx_ref, fW1_ref, fb1_ref, fW2_ref,
             fb2_ref, sW1x_ref, sW1z_ref, sb1_ref, sW2_ref, sb2_ref, sW3_ref,
             sb3_ref, out_ref, Sel_s, D_s, THi_s, TLo_s, *, B, N, M, H, RC):
    b = pl.program_id(0)
    K = _FRAME_KNN
    pn = pn_ref[0]
    pc = pc_ref[0]
    idx = idx_ref[...]

    pn_hi = pn.astype(jnp.bfloat16)
    pn_lo = (pn - pn_hi.astype(jnp.float32)).astype(jnp.bfloat16)
    pc_hi = pc.astype(jnp.bfloat16)
    pc_lo = (pc - pc_hi.astype(jnp.float32)).astype(jnp.bfloat16)
    pn9 = jnp.concatenate([pn_hi, pn_lo, pn_hi], axis=0)
    pc9 = jnp.concatenate([pc_hi, pc_lo, pc_hi], axis=0)

    def extract2(selb, rhs_hi, rhs_lo, dims):
        acc = jax.lax.dot_general(selb, rhs_hi, (dims, ((), ())),
                                  preferred_element_type=jnp.float32)
        return acc + jax.lax.dot_general(selb, rhs_lo, (dims, ((), ())),
                                         preferred_element_type=jnp.float32)

    def split(a):
        hi = a.astype(jnp.bfloat16)
        return hi, (a - hi.astype(jnp.float32)).astype(jnp.bfloat16)

    def cross3(ahi, alo, bstack):
        a9 = jnp.concatenate([ahi, ahi, alo], axis=1)
        return jax.lax.dot_general(a9, bstack, (((1,), (0,)), ((), ())),
                                   preferred_element_type=jnp.float32)

    iota = jax.lax.broadcasted_iota(jnp.int32, (M, N), 1)
    ohb = (iota == idx).astype(jnp.bfloat16)
    q = extract2(ohb, pn_hi, pn_lo, ((1,), (1,)))

    h = jnp.maximum(jnp.dot(q, fW1_ref[...]) + fb1_ref[...], 0.0)
    feat = jnp.dot(h, fW2_ref[...]) + fb2_ref[...]
    zpart = jnp.dot(feat, sW1z_ref[...])

    pcn = jnp.sum(pc * pc, axis=0, keepdims=True)
    ciota = jax.lax.broadcasted_iota(jnp.int32, (RC, N), 1)

    def clean_step(c, _):
        X = pnr_ref[0, pl.ds(c * RC, RC), :]
        xn = jnp.sum(X * X, axis=1, keepdims=True)
        xhi, xlo = split(X)
        cr = cross3(xhi, xlo, pc9)
        dd = jnp.maximum(xn + pcn - 2.0 * cr, 0.0)
        ik = jax.lax.bitcast_convert_type(dd, jnp.int32)
        ik = (ik & jnp.int32(-2048)) | ciota
        dd = jax.lax.bitcast_convert_type(ik, jnp.float32)
        m = jnp.min(dd, axis=1, keepdims=True)
        for _i in range(_NUM_CLEAN_NBS - 1):
            m = jnp.min(jnp.where(dd > m, dd, _BIG), axis=1, keepdims=True)
        mask = (dd <= m).astype(jnp.bfloat16)
        s4 = extract2(mask, pc_hi, pc_lo, ((1,), (1,)))
        tgt = s4 * (1.0 / _NUM_CLEAN_NBS) - X
        val = jnp.concatenate([X, tgt], axis=1)
        vhi, vlo = split(val)
        THi_s[pl.ds(c * RC, RC), :] = vhi
        TLo_s[pl.ds(c * RC, RC), :] = vlo
        return 0

    qn = jnp.sum(q * q, axis=1, keepdims=True)
    pnn = jnp.sum(pn * pn, axis=0, keepdims=True)
    qhi, qlo = split(q)
    cross = cross3(qhi, qlo, pn9)
    d2q = jnp.maximum(qn + pnn - 2.0 * cross, 0.0)
    ikq = jax.lax.bitcast_convert_type(d2q, jnp.int32)
    ikq = (ikq & jnp.int32(-2048)) | iota
    D_s[0:M, :] = jax.lax.bitcast_convert_type(ikq, jnp.float32)

    PER = K // (N // RC)

    def fused_step(j, _):
        dd = D_s[0:M, :]
        for t in range(PER):
            mn = jnp.min(dd, axis=1, keepdims=True)
            sel = (dd == mn)
            Sel_s[pl.ds((j * PER + t) * M, M), :] = sel.astype(jnp.bfloat16)
            dd = jnp.where(sel, _BIG, dd)
        D_s[0:M, :] = dd
        clean_step(j, _)
        return 0

    jax.lax.fori_loop(0, N // RC, fused_step, 0)

    ext = extract2(Sel_s[...], THi_s[...], TLo_s[...], ((1,), (0,)))

    Fa = ext[:, 0:3]
    Qt = jnp.broadcast_to(q[None], (K, M, 3)).reshape(K * M, 3)
    XC = Fa - Qt
    Zt = jnp.broadcast_to(zpart[None], (K, M, H)).reshape(K * M, H)
    h1 = jnp.dot(XC, sW1x_ref[...]) + Zt + sb1_ref[...]
    h1 = jnp.maximum(h1, 0.0)
    h2 = jnp.maximum(jnp.dot(h1, sW2_ref[...]) + sb2_ref[...], 0.0)
    g = jax.lax.dot_general(h2, sW3_ref[...],
                            (((1,), (0,)), ((), ()))) + sb3_ref[...]

    diff = ext[:, 3:6] - g
    s = jnp.sum(diff * diff)

    @pl.when(b == 0)
    def _init():
        out_ref[0, 0] = 0.0

    out_ref[0, 0] = out_ref[0, 0] + s

    @pl.when(b == B - 1)
    def _fin():
        scale = 0.5 * (1.0 / _DSM_SIGMA) / (B * M * K)
        out_ref[0, 0] = out_ref[0, 0] * scale


def kernel(pcl_noisy, pcl_clean, pnt_idx, fW1, fb1, fW2, fb2,
           sW1, sb1, sW2, sb2, sW3, sb3):
    B, N, d = pcl_noisy.shape
    M = pnt_idx.shape[0]
    H = fW1.shape[1]
    K = _FRAME_KNN
    RC = 256

    pn_t = jnp.transpose(pcl_noisy, (0, 2, 1))
    pc_t = jnp.transpose(pcl_clean, (0, 2, 1))
    idx2 = pnt_idx.reshape(M, 1).astype(jnp.int32)

    body = functools.partial(_dn_body, B=B, N=N, M=M, H=H, RC=RC)
    bspec = lambda blk, imap: pl.BlockSpec(blk, imap)
    rep = lambda *shape: pl.BlockSpec(shape, lambda b: (0,) * len(shape))

    res = pl.pallas_call(
        body,
        grid=(B,),
        in_specs=[
            bspec((1, 3, N), lambda b: (b, 0, 0)),
            bspec((1, 3, N), lambda b: (b, 0, 0)),
            bspec((1, N, 3), lambda b: (b, 0, 0)),
            rep(M, 1),
            rep(3, H), rep(1, H), rep(H, H), rep(1, H),
            rep(3, H), rep(H, H), rep(1, H),
            rep(H, H), rep(1, H), rep(H, 3), rep(1, 3),
        ],
        out_specs=pl.BlockSpec((1, 1), lambda b: (0, 0),
                               memory_space=pltpu.SMEM),
        out_shape=jax.ShapeDtypeStruct((1, 1), jnp.float32),
        scratch_shapes=[
            pltpu.VMEM((M * K, N), jnp.bfloat16),
            pltpu.VMEM((M, N), jnp.float32),
            pltpu.VMEM((N, 6), jnp.bfloat16),
            pltpu.VMEM((N, 6), jnp.bfloat16),
        ],
    )(pn_t, pc_t, pcl_noisy, idx2,
      fW1, fb1.reshape(1, H), fW2, fb2.reshape(1, H),
      sW1[:d], sW1[d:], sb1.reshape(1, H),
      sW2, sb2.reshape(1, H), sW3, sb3.reshape(1, 3))
    return res[0, 0]


# speedup vs baseline: 91.6872x; 1.0625x over previous
import functools

import jax
import jax.numpy as jnp
from jax.experimental import pallas as pl
from jax.experimental.pallas import tpu as pltpu

_FRAME_KNN = 32
_NUM_CLEAN_NBS = 4
_DSM_SIGMA = 0.01
_HIGH = jax.lax.Precision.HIGHEST
_BIG = 3.0e38


def _dn_body(pn_ref, pc_ref, pnr_ref, idx_ref, fW1_ref, fb1_ref, fW2_ref,
             fb2_ref, sW1x_ref, sW1z_ref, sb1_ref, sW2_ref, sb2_ref, sW3_ref,
             sb3_ref, out_ref, Sel_s, D_s, THi_s, TLo_s, *, B, N, M, H, RC):
    b = pl.program_id(0)
    K = _FRAME_KNN
    pn = pn_ref[0]
    pc = pc_ref[0]
    idx = idx_ref[...]

    pn_hi = pn.astype(jnp.bfloat16)
    pn_lo = (pn - pn_hi.astype(jnp.float32)).astype(jnp.bfloat16)
    pc_hi = pc.astype(jnp.bfloat16)
    pc_lo = (pc - pc_hi.astype(jnp.float32)).astype(jnp.bfloat16)
    pn9 = jnp.concatenate([pn_hi, pn_lo, pn_hi], axis=0)
    pc9 = jnp.concatenate([pc_hi, pc_lo, pc_hi], axis=0)

    def extract2(selb, rhs_hi, rhs_lo, dims):
        acc = jax.lax.dot_general(selb, rhs_hi, (dims, ((), ())),
                                  preferred_element_type=jnp.float32)
        return acc + jax.lax.dot_general(selb, rhs_lo, (dims, ((), ())),
                                         preferred_element_type=jnp.float32)

    def split(a):
        hi = a.astype(jnp.bfloat16)
        return hi, (a - hi.astype(jnp.float32)).astype(jnp.bfloat16)

    def cross3(ahi, alo, bstack):
        a9 = jnp.concatenate([ahi, ahi, alo], axis=1)
        return jax.lax.dot_general(a9, bstack, (((1,), (0,)), ((), ())),
                                   preferred_element_type=jnp.float32)

    iota = jax.lax.broadcasted_iota(jnp.int32, (M, N), 1)
    ohb = (iota == idx).astype(jnp.bfloat16)
    q = extract2(ohb, pn_hi, pn_lo, ((1,), (1,)))

    h = jnp.maximum(jnp.dot(q, fW1_ref[...]) + fb1_ref[...], 0.0)
    feat = jnp.dot(h, fW2_ref[...]) + fb2_ref[...]
    zpart = jnp.dot(feat, sW1z_ref[...])

    pcn = jnp.sum(pc * pc, axis=0, keepdims=True)
    ciota = jax.lax.broadcasted_iota(jnp.int32, (RC, N), 1)

    def clean_step(c, _):
        X = pnr_ref[0, pl.ds(c * RC, RC), :]
        xn = jnp.sum(X * X, axis=1, keepdims=True)
        xhi, xlo = split(X)
        cr = cross3(xhi, xlo, pc9)
        dd = jnp.maximum(xn + pcn - 2.0 * cr, 0.0)
        ik = jax.lax.bitcast_convert_type(dd, jnp.int32)
        ik = (ik & jnp.int32(-2048)) | ciota
        dd = jax.lax.bitcast_convert_type(ik, jnp.float32)
        m = jnp.min(dd, axis=1, keepdims=True)
        for _i in range(_NUM_CLEAN_NBS - 1):
            m = jnp.min(jnp.where(dd > m, dd, _BIG), axis=1, keepdims=True)
        mask = (dd <= m).astype(jnp.bfloat16)
        s4 = extract2(mask, pc_hi, pc_lo, ((1,), (1,)))
        tgt = s4 * (1.0 / _NUM_CLEAN_NBS) - X
        val = jnp.concatenate([X, tgt], axis=1)
        vhi, vlo = split(val)
        THi_s[pl.ds(c * RC, RC), :] = vhi
        TLo_s[pl.ds(c * RC, RC), :] = vlo
        return 0

    qn = jnp.sum(q * q, axis=1, keepdims=True)
    pnn = jnp.sum(pn * pn, axis=0, keepdims=True)
    qhi, qlo = split(q)
    cross = cross3(qhi, qlo, pn9)
    d2q = jnp.maximum(qn + pnn - 2.0 * cross, 0.0)
    ikq = jax.lax.bitcast_convert_type(d2q, jnp.int32)
    ikq = (ikq & jnp.int32(-2048)) | iota
    D_s[0:M, :] = jax.lax.bitcast_convert_type(ikq, jnp.float32)

    PER = K // (N // RC)

    def fused_step(j, _):
        dd = D_s[0:M, :]
        for t in range(PER):
            mn = jnp.min(dd, axis=1, keepdims=True)
            sel = (dd == mn)
            Sel_s[pl.ds((j * PER + t) * M, M), :] = sel.astype(jnp.bfloat16)
            dd = jnp.where(sel, _BIG, dd)
        D_s[0:M, :] = dd
        clean_step(j, _)
        return 0

    jax.lax.fori_loop(0, N // RC, fused_step, 0)

    ext = extract2(Sel_s[...], THi_s[...], TLo_s[...], ((1,), (0,)))

    Fa = ext[:, 0:3]
    Qt = jnp.broadcast_to(q[None], (K, M, 3)).reshape(K * M, 3)
    XC = Fa - Qt
    Zt = jnp.broadcast_to(zpart[None], (K, M, H)).reshape(K * M, H)
    h1 = jnp.dot(XC, sW1x_ref[...]) + Zt + sb1_ref[...]
    h1 = jnp.maximum(h1, 0.0)
    h2 = jnp.maximum(jnp.dot(h1, sW2_ref[...]) + sb2_ref[...], 0.0)
    g = jax.lax.dot_general(h2, sW3_ref[...],
                            (((1,), (0,)), ((), ()))) + sb3_ref[...]

    diff = ext[:, 3:6] - g
    s = jnp.sum(diff * diff)

    @pl.when(b == 0)
    def _init():
        out_ref[0, 0] = 0.0

    out_ref[0, 0] = out_ref[0, 0] + s

    @pl.when(b == B - 1)
    def _fin():
        scale = 0.5 * (1.0 / _DSM_SIGMA) / (B * M * K)
        out_ref[0, 0] = out_ref[0, 0] * scale


def kernel(pcl_noisy, pcl_clean, pnt_idx, fW1, fb1, fW2, fb2,
           sW1, sb1, sW2, sb2, sW3, sb3):
    B, N, d = pcl_noisy.shape
    M = pnt_idx.shape[0]
    H = fW1.shape[1]
    K = _FRAME_KNN
    RC = 512

    pn_t = jnp.transpose(pcl_noisy, (0, 2, 1))
    pc_t = jnp.transpose(pcl_clean, (0, 2, 1))
    idx2 = pnt_idx.reshape(M, 1).astype(jnp.int32)

    body = functools.partial(_dn_body, B=B, N=N, M=M, H=H, RC=RC)
    bspec = lambda blk, imap: pl.BlockSpec(blk, imap)
    rep = lambda *shape: pl.BlockSpec(shape, lambda b: (0,) * len(shape))

    res = pl.pallas_call(
        body,
        grid=(B,),
        in_specs=[
            bspec((1, 3, N), lambda b: (b, 0, 0)),
            bspec((1, 3, N), lambda b: (b, 0, 0)),
            bspec((1, N, 3), lambda b: (b, 0, 0)),
            rep(M, 1),
            rep(3, H), rep(1, H), rep(H, H), rep(1, H),
            rep(3, H), rep(H, H), rep(1, H),
            rep(H, H), rep(1, H), rep(H, 3), rep(1, 3),
        ],
        out_specs=pl.BlockSpec((1, 1), lambda b: (0, 0),
                               memory_space=pltpu.SMEM),
        out_shape=jax.ShapeDtypeStruct((1, 1), jnp.float32),
        scratch_shapes=[
            pltpu.VMEM((M * K, N), jnp.bfloat16),
            pltpu.VMEM((M, N), jnp.float32),
            pltpu.VMEM((N, 6), jnp.bfloat16),
            pltpu.VMEM((N, 6), jnp.bfloat16),
        ],
    )(pn_t, pc_t, pcl_noisy, idx2,
      fW1, fb1.reshape(1, H), fW2, fb2.reshape(1, H),
      sW1[:d], sW1[d:], sb1.reshape(1, H),
      sW2, sb2.reshape(1, H), sW3, sb3.reshape(1, 3))
    return res[0, 0]
